# Initial kernel scaffold; baseline (speedup 1.0000x reference)
import jax
import jax.numpy as jnp
from jax.experimental import pallas as pl


def kernel(x, edge_index, W1, b1, W2, b2, W3, b3):
    raise NotImplementedError("write your pallas kernel here")


# speedup vs baseline: 24.3270x; 24.3270x over previous
import functools

import jax
import jax.numpy as jnp
from jax import lax
from jax.experimental import pallas as pl
from jax.experimental.pallas import tpu as pltpu
from jax.experimental.pallas import tpu_sc as plsc

NC = 2
NS = 16
CHUNK = 128


def _sc_degree(np_rows, e_pad):
    epw = e_pad // (NC * NS)
    nch = epw // CHUNK
    rpt = np_rows // NS
    mesh = plsc.VectorSubcoreMesh(core_axis_name="c", subcore_axis_name="s")

    @functools.partial(
        pl.kernel,
        out_type=jax.ShapeDtypeStruct((NC, np_rows), jnp.float32),
        mesh=mesh,
        scratch_types=[
            pltpu.VMEM((nch, CHUNK), jnp.int32),
            pltpu.VMEM((CHUNK,), jnp.float32),
            pltpu.VMEM((rpt,), jnp.float32),
            pltpu.VMEM_SHARED((np_rows,), jnp.float32),
        ],
    )
    def deg_kernel(dst_hbm, out_hbm, didx, ones, zbuf, acc):
        c = lax.axis_index("c")
        s = lax.axis_index("s")
        wid = c * NS + s
        row0 = s * rpt

        def fill(i, _):
            ones[pl.ds(i * 16, 16)] = jnp.ones((16,), jnp.float32)
            return _
        lax.fori_loop(0, CHUNK // 16, fill, None)

        def zfill(i, _):
            zbuf[pl.ds(i * 16, 16)] = jnp.zeros((16,), jnp.float32)
            return _
        lax.fori_loop(0, rpt // 16, zfill, None)
        pltpu.sync_copy(zbuf, acc.at[pl.ds(row0, rpt)])
        plsc.subcore_barrier()

        pltpu.sync_copy(dst_hbm.at[pl.ds(wid * nch, nch)], didx)

        def body(j, _):
            pltpu.sync_copy(ones, acc.at[didx.at[j]], add=True)
            return _
        lax.fori_loop(0, nch, body, None)
        plsc.subcore_barrier()

        pltpu.sync_copy(acc.at[pl.ds(row0, rpt)],
                        out_hbm.at[c, pl.ds(row0, rpt)])

    return deg_kernel


def _sc_aggregate(np_rows, e_pad, d):
    epw = e_pad // (NC * NS)
    nch = epw // CHUNK
    rpt = np_rows // NS
    mesh = plsc.VectorSubcoreMesh(core_axis_name="c", subcore_axis_name="s")

    @functools.partial(
        pl.kernel,
        out_type=jax.ShapeDtypeStruct((NC, np_rows, d), jnp.float32),
        mesh=mesh,
        compiler_params=pltpu.CompilerParams(use_tc_tiling_on_sc=False),
        scratch_types=[
            pltpu.VMEM((nch, CHUNK), jnp.int32),
            pltpu.VMEM((nch, CHUNK), jnp.int32),
            pltpu.VMEM((2, CHUNK, d), jnp.float32),
            pltpu.VMEM((rpt, d), jnp.float32),
            pltpu.VMEM_SHARED((np_rows, d), jnp.float32),
            pltpu.SemaphoreType.DMA,
        ],
    )
    def agg_kernel(src_hbm, dst_hbm, table_hbm, out_hbm,
                   sidx, didx, rows, zbuf, acc, sem0):
        c = lax.axis_index("c")
        s = lax.axis_index("s")
        wid = c * NS + s
        row0 = s * rpt

        kd = max(d // 16, 1)

        def zfill(t, _):
            r = t // kd
            k = t % kd
            zbuf[r, pl.ds(k * 16, 16)] = jnp.zeros((16,), jnp.float32)
            return _
        lax.fori_loop(0, rpt * d // 16, zfill, None)
        pltpu.sync_copy(zbuf, acc.at[pl.ds(row0, rpt)])
        plsc.subcore_barrier()

        pltpu.sync_copy(src_hbm.at[pl.ds(wid * nch, nch)], sidx)
        pltpu.sync_copy(dst_hbm.at[pl.ds(wid * nch, nch)], didx)

        def body(j, _):
            pltpu.async_copy(table_hbm.at[sidx.at[j]], rows.at[0],
                             sem0).wait()
            pltpu.sync_copy(rows.at[0], acc.at[didx.at[j]], add=True)
            return _
        lax.fori_loop(0, nch, body, None)
        plsc.subcore_barrier()

        pltpu.sync_copy(acc.at[pl.ds(row0, rpt)],
                        out_hbm.at[c, pl.ds(row0, rpt)])

    return agg_kernel


def _sc_aggregate_1d(np_rows, e_pad):
    epw = e_pad // (NC * NS)
    nch = epw // CHUNK
    rpt = np_rows // NS
    mesh = plsc.VectorSubcoreMesh(core_axis_name="c", subcore_axis_name="s")

    @functools.partial(
        pl.kernel,
        out_type=jax.ShapeDtypeStruct((NC, np_rows), jnp.float32),
        mesh=mesh,
        scratch_types=[
            pltpu.VMEM((nch, CHUNK), jnp.int32),
            pltpu.VMEM((nch, CHUNK), jnp.int32),
            pltpu.VMEM((CHUNK,), jnp.float32),
            pltpu.VMEM((rpt,), jnp.float32),
            pltpu.VMEM_SHARED((np_rows,), jnp.float32),
            pltpu.SemaphoreType.DMA,
        ],
    )
    def agg1d_kernel(src_hbm, dst_hbm, table_hbm, out_hbm,
                     sidx, didx, rows, zbuf, acc, sem0):
        c = lax.axis_index("c")
        s = lax.axis_index("s")
        wid = c * NS + s
        row0 = s * rpt

        def zfill(i, _):
            zbuf[pl.ds(i * 16, 16)] = jnp.zeros((16,), jnp.float32)
            return _
        lax.fori_loop(0, rpt // 16, zfill, None)
        pltpu.sync_copy(zbuf, acc.at[pl.ds(row0, rpt)])
        plsc.subcore_barrier()

        pltpu.sync_copy(src_hbm.at[pl.ds(wid * nch, nch)], sidx)
        pltpu.sync_copy(dst_hbm.at[pl.ds(wid * nch, nch)], didx)

        def body(j, _):
            pltpu.async_copy(table_hbm.at[sidx.at[j]], rows, sem0).wait()
            pltpu.sync_copy(rows, acc.at[didx.at[j]], add=True)
            return _
        lax.fori_loop(0, nch, body, None)
        plsc.subcore_barrier()

        pltpu.sync_copy(acc.at[pl.ds(row0, rpt)],
                        out_hbm.at[c, pl.ds(row0, rpt)])

    return agg1d_kernel


def _tc_first(np_rows, n, d_in, d_out, blk):
    grid = (np_rows // blk,)

    def body(deg_ref, x_ref, w_ref, dis_ref, scaled_ref):
        i = pl.program_id(0)
        deg = deg_ref[0] + deg_ref[1] + 1.0
        dis = lax.rsqrt(deg)
        row = jax.lax.broadcasted_iota(jnp.int32, (blk, 1), 0) + i * blk
        dis = jnp.where(row < n, dis, 0.0)
        pre = jnp.dot(x_ref[...], w_ref[...],
                      preferred_element_type=jnp.float32)
        dis_ref[...] = dis
        scaled_ref[...] = jnp.where(row < n, dis * pre, 0.0)

    return pl.pallas_call(
        body,
        grid=grid,
        in_specs=[
            pl.BlockSpec((2, blk, 1), lambda i: (0, i, 0)),
            pl.BlockSpec((blk, d_in), lambda i: (i, 0)),
            pl.BlockSpec((d_in, d_out), lambda i: (0, 0)),
        ],
        out_specs=[
            pl.BlockSpec((blk, 1), lambda i: (i, 0)),
            pl.BlockSpec((blk, d_out), lambda i: (i, 0)),
        ],
        out_shape=[
            jax.ShapeDtypeStruct((np_rows, 1), jnp.float32),
            jax.ShapeDtypeStruct((np_rows, d_out), jnp.float32),
        ],
    )


def _tc_mid(np_rows, d, d_next, blk):
    grid = (np_rows // blk,)

    def body(agg_ref, scaled_ref, dis_ref, b_ref, w_ref, out_ref):
        a = agg_ref[0] + agg_ref[1] + scaled_ref[...]
        h = jnp.maximum(dis_ref[...] * a + b_ref[...], 0.0)
        out_ref[...] = dis_ref[...] * jnp.dot(
            h, w_ref[...], preferred_element_type=jnp.float32)

    return pl.pallas_call(
        body,
        grid=grid,
        in_specs=[
            pl.BlockSpec((2, blk, d), lambda i: (0, i, 0)),
            pl.BlockSpec((blk, d), lambda i: (i, 0)),
            pl.BlockSpec((blk, 1), lambda i: (i, 0)),
            pl.BlockSpec((1, d), lambda i: (0, 0)),
            pl.BlockSpec((d, d_next), lambda i: (0, 0)),
        ],
        out_specs=pl.BlockSpec((blk, d_next), lambda i: (i, 0)),
        out_shape=jax.ShapeDtypeStruct((np_rows, d_next), jnp.float32),
    )


def _tc_last(np_rows, n, blk):
    grid = (np_rows // blk,)

    def body(agg_ref, scaled_ref, dis_ref, b_ref, out_ref):
        a = agg_ref[0] + agg_ref[1] + scaled_ref[...]
        out_ref[...] = dis_ref[...] * a + b_ref[...]

    return pl.pallas_call(
        body,
        grid=grid,
        in_specs=[
            pl.BlockSpec((2, blk, 1), lambda i: (0, i, 0)),
            pl.BlockSpec((blk, 1), lambda i: (i, 0)),
            pl.BlockSpec((blk, 1), lambda i: (i, 0)),
            pl.BlockSpec((1, 1), lambda i: (0, 0)),
        ],
        out_specs=pl.BlockSpec((blk, 1), lambda i: (i, 0)),
        out_shape=jax.ShapeDtypeStruct((n, 1), jnp.float32),
    )


@jax.jit
def kernel(x, edge_index, W1, b1, W2, b2, W3, b3):
    n, d_in = x.shape
    e = edge_index.shape[1]
    d1 = W1.shape[1]
    d2 = W2.shape[1]

    blk = 1024
    np_rows = ((n + 1 + blk - 1) // blk) * blk
    unit = NC * NS * CHUNK * 8
    e_pad = ((e + unit - 1) // unit) * unit
    pad = e_pad - e
    fillv = jnp.full((pad,), n, jnp.int32)
    src2d = jnp.concatenate([edge_index[0], fillv]).reshape(-1, CHUNK)
    dst2d = jnp.concatenate([edge_index[1], fillv]).reshape(-1, CHUNK)

    deg2 = _sc_degree(np_rows, e_pad)(dst2d)
    dis, scaled1 = _tc_first(np_rows, n, d_in, d1, blk)(
        deg2.reshape(NC, np_rows, 1), x, W1)
    agg1 = _sc_aggregate(np_rows, e_pad, d1)(src2d, dst2d, scaled1)
    scaled2 = _tc_mid(np_rows, d1, d2, blk)(
        agg1, scaled1, dis, b1.reshape(1, -1), W2)
    agg2 = _sc_aggregate(np_rows, e_pad, d2)(src2d, dst2d, scaled2)
    scaled3 = _tc_mid(np_rows, d2, W3.shape[1], blk)(
        agg2, scaled2, dis, b2.reshape(1, -1), W3)
    agg3 = _sc_aggregate_1d(np_rows, e_pad)(
        src2d, dst2d, scaled3.reshape(np_rows))
    out = _tc_last(np_rows, n, blk)(
        agg3.reshape(NC, np_rows, 1), scaled3, dis, b3.reshape(1, 1))
    return out


# speedup vs baseline: 31.3968x; 1.2906x over previous
import functools

import jax
import jax.numpy as jnp
from jax import lax
from jax.experimental import pallas as pl
from jax.experimental.pallas import tpu as pltpu
from jax.experimental.pallas import tpu_sc as plsc

NC = 2
NS = 16
CHUNK = 128


def _sc_degree(np_rows, e_pad):
    epw = e_pad // (NC * NS)
    nch = epw // CHUNK
    rpt = np_rows // NS
    mesh = plsc.VectorSubcoreMesh(core_axis_name="c", subcore_axis_name="s")

    @functools.partial(
        pl.kernel,
        out_type=jax.ShapeDtypeStruct((NC, np_rows), jnp.float32),
        mesh=mesh,
        scratch_types=[
            pltpu.VMEM((nch, CHUNK), jnp.int32),
            pltpu.VMEM((CHUNK,), jnp.float32),
            pltpu.VMEM((rpt,), jnp.float32),
            pltpu.VMEM_SHARED((np_rows,), jnp.float32),
            pltpu.SemaphoreType.DMA,
        ],
    )
    def deg_kernel(dst_hbm, out_hbm, didx, ones, zbuf, acc, sem0):
        c = lax.axis_index("c")
        s = lax.axis_index("s")
        wid = c * NS + s
        row0 = s * rpt

        def fill(i, _):
            ones[pl.ds(i * 16, 16)] = jnp.ones((16,), jnp.float32)
            return _
        lax.fori_loop(0, CHUNK // 16, fill, None)

        def zfill(i, _):
            zbuf[pl.ds(i * 16, 16)] = jnp.zeros((16,), jnp.float32)
            return _
        lax.fori_loop(0, rpt // 16, zfill, None)
        pltpu.sync_copy(zbuf, acc.at[pl.ds(row0, rpt)])
        plsc.subcore_barrier()

        pltpu.sync_copy(dst_hbm.at[pl.ds(wid * nch, nch)], didx)

        k = 16

        def outer(t, _):
            def fire(i, _):
                pltpu.async_copy(ones, acc.at[didx.at[t * k + i]], sem0,
                                 add=True)
                return _
            lax.fori_loop(0, k, fire, None)

            def drain(i, _):
                pltpu.make_async_copy(ones, acc.at[didx.at[t * k + i]],
                                      sem0).wait()
                return _
            lax.fori_loop(0, k, drain, None)
            return _
        lax.fori_loop(0, nch // k, outer, None)
        plsc.subcore_barrier()

        pltpu.sync_copy(acc.at[pl.ds(row0, rpt)],
                        out_hbm.at[c, pl.ds(row0, rpt)])

    return deg_kernel


def _sc_aggregate(np_rows, e_pad, d):
    epw = e_pad // (NC * NS)
    nch = epw // CHUNK
    rpt = np_rows // NS
    mesh = plsc.VectorSubcoreMesh(core_axis_name="c", subcore_axis_name="s")

    nbuf = 4
    koff = 2

    @functools.partial(
        pl.kernel,
        out_type=jax.ShapeDtypeStruct((NC, np_rows, d), jnp.float32),
        mesh=mesh,
        compiler_params=pltpu.CompilerParams(use_tc_tiling_on_sc=False),
        scratch_types=[
            pltpu.VMEM((nch, CHUNK), jnp.int32),
            pltpu.VMEM((nch, CHUNK), jnp.int32),
            pltpu.VMEM((nbuf, CHUNK, d), jnp.float32),
            pltpu.VMEM((CHUNK, d), jnp.float32),
            pltpu.VMEM_SHARED((np_rows, d), jnp.float32),
        ] + [pltpu.SemaphoreType.DMA] * (2 * nbuf),
    )
    def agg_kernel(src_hbm, dst_hbm, table_hbm, out_hbm,
                   sidx, didx, rows, zbuf, acc, *sems):
        gsem = sems[:nbuf]
        ssem = sems[nbuf:]
        c = lax.axis_index("c")
        s = lax.axis_index("s")
        wid = c * NS + s
        row0 = s * rpt

        kd = max(d // 16, 1)

        def zfill(t, _):
            r = t // kd
            k = t % kd
            zbuf[r, pl.ds(k * 16, 16)] = jnp.zeros((16,), jnp.float32)
            return _
        lax.fori_loop(0, CHUNK * d // 16, zfill, None)
        for i in range(rpt // CHUNK):
            pltpu.sync_copy(zbuf, acc.at[pl.ds(row0 + i * CHUNK, CHUNK)])
        plsc.subcore_barrier()

        pltpu.sync_copy(src_hbm.at[pl.ds(wid * nch, nch)], sidx)
        pltpu.sync_copy(dst_hbm.at[pl.ds(wid * nch, nch)], didx)

        for b in range(koff):
            pltpu.async_copy(table_hbm.at[sidx.at[b]], rows.at[b], gsem[b])

        def outer(t, _):
            for b in range(nbuf):
                j = t * nbuf + b
                pltpu.make_async_copy(table_hbm.at[sidx.at[j]], rows.at[b],
                                      gsem[b]).wait()
                pltpu.async_copy(rows.at[b], acc.at[didx.at[j]], ssem[b],
                                 add=True)
                jn = j + koff
                bn = (b + koff) % nbuf

                @pl.when(jn < nch)
                def _start_next():
                    @pl.when(j >= nbuf - koff)
                    def _wait_prev_scatter():
                        pltpu.make_async_copy(
                            rows.at[bn], acc.at[didx.at[jn - nbuf]],
                            ssem[bn]).wait()
                    pltpu.async_copy(table_hbm.at[sidx.at[jn]], rows.at[bn],
                                     gsem[bn])
            return _
        lax.fori_loop(0, nch // nbuf, outer, None)

        for jj in range(nch - nbuf, nch):
            b = jj % nbuf
            pltpu.make_async_copy(rows.at[b], acc.at[didx.at[jj]],
                                  ssem[b]).wait()
        plsc.subcore_barrier()

        pltpu.sync_copy(acc.at[pl.ds(row0, rpt)],
                        out_hbm.at[c, pl.ds(row0, rpt)])

    return agg_kernel


def _sc_aggregate_1d(np_rows, e_pad):
    epw = e_pad // (NC * NS)
    nch = epw // CHUNK
    rpt = np_rows // NS
    mesh = plsc.VectorSubcoreMesh(core_axis_name="c", subcore_axis_name="s")

    nbuf = 4
    koff = 2

    @functools.partial(
        pl.kernel,
        out_type=jax.ShapeDtypeStruct((NC, np_rows), jnp.float32),
        mesh=mesh,
        scratch_types=[
            pltpu.VMEM((nch, CHUNK), jnp.int32),
            pltpu.VMEM((nch, CHUNK), jnp.int32),
            pltpu.VMEM((nbuf, CHUNK), jnp.float32),
            pltpu.VMEM((rpt,), jnp.float32),
            pltpu.VMEM_SHARED((np_rows,), jnp.float32),
        ] + [pltpu.SemaphoreType.DMA] * (2 * nbuf),
    )
    def agg1d_kernel(src_hbm, dst_hbm, table_hbm, out_hbm,
                     sidx, didx, rows, zbuf, acc, *sems):
        gsem = sems[:nbuf]
        ssem = sems[nbuf:]
        c = lax.axis_index("c")
        s = lax.axis_index("s")
        wid = c * NS + s
        row0 = s * rpt

        def zfill(i, _):
            zbuf[pl.ds(i * 16, 16)] = jnp.zeros((16,), jnp.float32)
            return _
        lax.fori_loop(0, rpt // 16, zfill, None)
        pltpu.sync_copy(zbuf, acc.at[pl.ds(row0, rpt)])
        plsc.subcore_barrier()

        pltpu.sync_copy(src_hbm.at[pl.ds(wid * nch, nch)], sidx)
        pltpu.sync_copy(dst_hbm.at[pl.ds(wid * nch, nch)], didx)

        for b in range(koff):
            pltpu.async_copy(table_hbm.at[sidx.at[b]], rows.at[b], gsem[b])

        def outer(t, _):
            for b in range(nbuf):
                j = t * nbuf + b
                pltpu.make_async_copy(table_hbm.at[sidx.at[j]], rows.at[b],
                                      gsem[b]).wait()
                pltpu.async_copy(rows.at[b], acc.at[didx.at[j]], ssem[b],
                                 add=True)
                jn = j + koff
                bn = (b + koff) % nbuf

                @pl.when(jn < nch)
                def _start_next():
                    @pl.when(j >= nbuf - koff)
                    def _wait_prev_scatter():
                        pltpu.make_async_copy(
                            rows.at[bn], acc.at[didx.at[jn - nbuf]],
                            ssem[bn]).wait()
                    pltpu.async_copy(table_hbm.at[sidx.at[jn]], rows.at[bn],
                                     gsem[bn])
            return _
        lax.fori_loop(0, nch // nbuf, outer, None)

        for jj in range(nch - nbuf, nch):
            b = jj % nbuf
            pltpu.make_async_copy(rows.at[b], acc.at[didx.at[jj]],
                                  ssem[b]).wait()
        plsc.subcore_barrier()

        pltpu.sync_copy(acc.at[pl.ds(row0, rpt)],
                        out_hbm.at[c, pl.ds(row0, rpt)])

    return agg1d_kernel


def _tc_first(np_rows, n, d_in, d_out, blk):
    grid = (np_rows // blk,)

    def body(deg_ref, x_ref, w_ref, dis_ref, scaled_ref):
        i = pl.program_id(0)
        deg = deg_ref[0] + deg_ref[1] + 1.0
        dis = lax.rsqrt(deg)
        row = jax.lax.broadcasted_iota(jnp.int32, (blk, 1), 0) + i * blk
        dis = jnp.where(row < n, dis, 0.0)
        pre = jnp.dot(x_ref[...], w_ref[...],
                      preferred_element_type=jnp.float32)
        dis_ref[...] = dis
        scaled_ref[...] = jnp.where(row < n, dis * pre, 0.0)

    return pl.pallas_call(
        body,
        grid=grid,
        in_specs=[
            pl.BlockSpec((2, blk, 1), lambda i: (0, i, 0)),
            pl.BlockSpec((blk, d_in), lambda i: (i, 0)),
            pl.BlockSpec((d_in, d_out), lambda i: (0, 0)),
        ],
        out_specs=[
            pl.BlockSpec((blk, 1), lambda i: (i, 0)),
            pl.BlockSpec((blk, d_out), lambda i: (i, 0)),
        ],
        out_shape=[
            jax.ShapeDtypeStruct((np_rows, 1), jnp.float32),
            jax.ShapeDtypeStruct((np_rows, d_out), jnp.float32),
        ],
    )


def _tc_mid(np_rows, d, d_next, blk):
    grid = (np_rows // blk,)

    def body(agg_ref, scaled_ref, dis_ref, b_ref, w_ref, out_ref):
        a = agg_ref[0] + agg_ref[1] + scaled_ref[...]
        h = jnp.maximum(dis_ref[...] * a + b_ref[...], 0.0)
        out_ref[...] = dis_ref[...] * jnp.dot(
            h, w_ref[...], preferred_element_type=jnp.float32)

    return pl.pallas_call(
        body,
        grid=grid,
        in_specs=[
            pl.BlockSpec((2, blk, d), lambda i: (0, i, 0)),
            pl.BlockSpec((blk, d), lambda i: (i, 0)),
            pl.BlockSpec((blk, 1), lambda i: (i, 0)),
            pl.BlockSpec((1, d), lambda i: (0, 0)),
            pl.BlockSpec((d, d_next), lambda i: (0, 0)),
        ],
        out_specs=pl.BlockSpec((blk, d_next), lambda i: (i, 0)),
        out_shape=jax.ShapeDtypeStruct((np_rows, d_next), jnp.float32),
    )


def _tc_last(np_rows, n, blk):
    grid = (np_rows // blk,)

    def body(agg_ref, scaled_ref, dis_ref, b_ref, out_ref):
        a = agg_ref[0] + agg_ref[1] + scaled_ref[...]
        out_ref[...] = dis_ref[...] * a + b_ref[...]

    return pl.pallas_call(
        body,
        grid=grid,
        in_specs=[
            pl.BlockSpec((2, blk, 1), lambda i: (0, i, 0)),
            pl.BlockSpec((blk, 1), lambda i: (i, 0)),
            pl.BlockSpec((blk, 1), lambda i: (i, 0)),
            pl.BlockSpec((1, 1), lambda i: (0, 0)),
        ],
        out_specs=pl.BlockSpec((blk, 1), lambda i: (i, 0)),
        out_shape=jax.ShapeDtypeStruct((n, 1), jnp.float32),
    )


@jax.jit
def kernel(x, edge_index, W1, b1, W2, b2, W3, b3):
    n, d_in = x.shape
    e = edge_index.shape[1]
    d1 = W1.shape[1]
    d2 = W2.shape[1]

    blk = 1024
    np_rows = ((n + 1 + blk - 1) // blk) * blk
    unit = NC * NS * CHUNK * 8
    e_pad = ((e + unit - 1) // unit) * unit
    pad = e_pad - e
    fillv = jnp.full((pad,), n, jnp.int32)
    src2d = jnp.concatenate([edge_index[0], fillv]).reshape(-1, CHUNK)
    dst2d = jnp.concatenate([edge_index[1], fillv]).reshape(-1, CHUNK)

    deg2 = _sc_degree(np_rows, e_pad)(dst2d)
    dis, scaled1 = _tc_first(np_rows, n, d_in, d1, blk)(
        deg2.reshape(NC, np_rows, 1), x, W1)
    agg1 = _sc_aggregate(np_rows, e_pad, d1)(src2d, dst2d, scaled1)
    scaled2 = _tc_mid(np_rows, d1, d2, blk)(
        agg1, scaled1, dis, b1.reshape(1, -1), W2)
    agg2 = _sc_aggregate(np_rows, e_pad, d2)(src2d, dst2d, scaled2)
    scaled3 = _tc_mid(np_rows, d2, W3.shape[1], blk)(
        agg2, scaled2, dis, b2.reshape(1, -1), W3)
    agg3 = _sc_aggregate_1d(np_rows, e_pad)(
        src2d, dst2d, scaled3.reshape(np_rows))
    out = _tc_last(np_rows, n, blk)(
        agg3.reshape(NC, np_rows, 1), scaled3, dis, b3.reshape(1, 1))
    return out
